# Initial kernel scaffold; baseline (speedup 1.0000x reference)
import functools

import jax
import jax.numpy as jnp
from jax import lax
from jax.experimental import pallas as pl
from jax.experimental.pallas import tpu as pltpu
from jax.experimental.pallas import tpu_sc as plsc

N = 10000
NPAD = 10240
E = 320000
D = 128
DE = 16
NC = 2
NS = 16
NW = NC * NS
EPW = E // NW
BLK = 200
NBLK = EPW // BLK
CBLK = 1000
NCBLK = EPW // CBLK
STRIPE = NPAD // NS


def _lrelu(x):
    return jnp.where(x > 0, x, 0.2 * x)


def _node_mm_body(x_ref, w_ref, o0, o1, o2, o3):
    y = jnp.dot(x_ref[:], w_ref[:], preferred_element_type=jnp.float32)
    o0[:] = y[:, 0 * D:1 * D]
    o1[:] = y[:, 1 * D:2 * D]
    o2[:] = y[:, 2 * D:3 * D]
    o3[:] = y[:, 3 * D:4 * D]


def _node_transform(nf2d, wcat):
    blk = 2000
    sds = jax.ShapeDtypeStruct((N, D), jnp.float32)
    return pl.pallas_call(
        _node_mm_body,
        grid=(N // blk,),
        in_specs=[
            pl.BlockSpec((blk, D), lambda i: (i, 0)),
            pl.BlockSpec((D, 4 * D), lambda i: (0, 0)),
        ],
        out_specs=[pl.BlockSpec((blk, D), lambda i: (i, 0))] * 4,
        out_shape=[sds] * 4,
    )(nf2d, wcat)


def _edge_mm_body(x_ref, w_ref, b_ref, o0, o1):
    y = jnp.dot(x_ref[:], w_ref[:], preferred_element_type=jnp.float32)
    y = y + b_ref[:]
    o0[:] = y[:, 0 * D:1 * D]
    o1[:] = y[:, 1 * D:2 * D]


def _edge_transform(ef, wecat, becat):
    blk = 4000
    sds = jax.ShapeDtypeStruct((E, D), jnp.float32)
    return pl.pallas_call(
        _edge_mm_body,
        grid=(E // blk,),
        in_specs=[
            pl.BlockSpec((blk, DE), lambda i: (i, 0)),
            pl.BlockSpec((DE, 2 * D), lambda i: (0, 0)),
            pl.BlockSpec((1, 2 * D), lambda i: (0, 0)),
        ],
        out_specs=[pl.BlockSpec((blk, D), lambda i: (i, 0))] * 2,
        out_shape=[sds] * 2,
    )(ef, wecat, becat)


def _final_body(p0_ref, p1_ref, x_ref, w_ref, b_ref, o_ref):
    y = jnp.dot(x_ref[:], w_ref[:], preferred_element_type=jnp.float32)
    y = y + b_ref[:] + p0_ref[:] + p1_ref[:]
    o_ref[:] = _lrelu(y)


def _final_combine(p0, p1, nf2d, wnT, bn):
    blk = 2000
    return pl.pallas_call(
        _final_body,
        grid=(N // blk,),
        in_specs=[
            pl.BlockSpec((blk, D), lambda i: (i, 0)),
            pl.BlockSpec((blk, D), lambda i: (i, 0)),
            pl.BlockSpec((blk, D), lambda i: (i, 0)),
            pl.BlockSpec((D, D), lambda i: (0, 0)),
            pl.BlockSpec((1, D), lambda i: (0, 0)),
        ],
        out_specs=pl.BlockSpec((blk, D), lambda i: (i, 0)),
        out_shape=jax.ShapeDtypeStruct((N, D), jnp.float32),
    )(p0, p1, nf2d, wnT, bn)


_MESH = plsc.VectorSubcoreMesh(core_axis_name="c", subcore_axis_name="s")


def _worker_id():
    return lax.axis_index("s") * NC + lax.axis_index("c")


def _score_body(asrc, adst, eatt, uu, vv, wdot, scores,
                idxu, idxv, rows_u, rows_v, rows_e, wv, sbuf,
                semu, semv, seme):
    base = _worker_id() * EPW
    pltpu.sync_copy(wdot, wv)

    def blk_body(b, carry):
        off = base + b * BLK
        pltpu.sync_copy(uu.at[pl.ds(off, BLK)], idxu)
        pltpu.sync_copy(vv.at[pl.ds(off, BLK)], idxv)
        cu = pltpu.async_copy(asrc.at[idxu], rows_u, semu)
        cv = pltpu.async_copy(adst.at[idxv], rows_v, semv)
        ce = pltpu.async_copy(eatt.at[pl.ds(off, BLK)], rows_e, seme)
        cu.wait()
        cv.wait()
        ce.wait()

        def e_body(e, c2):
            acc = jnp.zeros((16,), jnp.float32)
            for d in range(D // 16):
                sl = pl.ds(d * 16, 16)
                x = rows_u[e, sl] + rows_v[e, sl] + rows_e[e, sl]
                acc = acc + wv[sl] * _lrelu(x)
            sbuf[e] = jnp.sum(acc)
            return c2

        lax.fori_loop(0, BLK, e_body, 0)
        pltpu.sync_copy(sbuf, scores.at[pl.ds(off, BLK)])
        return carry

    lax.fori_loop(0, NBLK, blk_body, 0)


def _scores(asrc, adst, eatt, uu, vv, wdot):
    f = pl.kernel(
        _score_body,
        out_type=jax.ShapeDtypeStruct((E,), jnp.float32),
        mesh=_MESH,
        scratch_types=[
            pltpu.VMEM((BLK,), jnp.int32),
            pltpu.VMEM((BLK,), jnp.int32),
            pltpu.VMEM((BLK, D), jnp.float32),
            pltpu.VMEM((BLK, D), jnp.float32),
            pltpu.VMEM((BLK, D), jnp.float32),
            pltpu.VMEM((D,), jnp.float32),
            pltpu.VMEM((BLK,), jnp.float32),
            pltpu.SemaphoreType.DMA,
            pltpu.SemaphoreType.DMA,
            pltpu.SemaphoreType.DMA,
        ],
    )
    return f(asrc, adst, eatt, uu, vv, wdot)


def _denom_body(scores, vv, ex, spart,
                vbuf, sbuf, ebuf, zbuf, obuf, acc):
    cid = lax.axis_index("c")
    sid = lax.axis_index("s")
    wid = sid * NC + cid
    base = wid * EPW

    def z_body(i, c):
        zbuf[pl.ds(i * 16, 16)] = jnp.zeros((16,), jnp.float32)
        return c

    lax.fori_loop(0, STRIPE // 16, z_body, 0)
    pltpu.sync_copy(zbuf, acc.at[pl.ds(sid * STRIPE, STRIPE)])
    plsc.subcore_barrier()

    def blk_body(b, carry):
        off = base + b * CBLK
        pltpu.sync_copy(vv.at[pl.ds(off, CBLK)], vbuf)
        pltpu.sync_copy(scores.at[pl.ds(off, CBLK)], sbuf)

        def g_body(g, c2):
            sl = pl.ds(g * 16, 16)
            ebuf[sl] = jnp.exp(sbuf[sl])
            return c2

        lax.fori_loop(0, CBLK // 16, g_body, 0)
        pltpu.sync_copy(ebuf, ex.at[pl.ds(off, CBLK)])
        pltpu.sync_copy(ebuf, acc.at[vbuf], add=True)
        return carry

    lax.fori_loop(0, NCBLK, blk_body, 0)
    plsc.subcore_barrier()
    pltpu.sync_copy(acc.at[pl.ds(sid * STRIPE, STRIPE)], obuf)
    pltpu.sync_copy(obuf, spart.at[cid, pl.ds(sid * STRIPE, STRIPE)])


def _denoms(scores, vv):
    f = pl.kernel(
        _denom_body,
        out_type=(
            jax.ShapeDtypeStruct((E,), jnp.float32),
            jax.ShapeDtypeStruct((NC, NPAD), jnp.float32),
        ),
        mesh=_MESH,
        scratch_types=[
            pltpu.VMEM((CBLK,), jnp.int32),
            pltpu.VMEM((CBLK,), jnp.float32),
            pltpu.VMEM((CBLK,), jnp.float32),
            pltpu.VMEM((STRIPE,), jnp.float32),
            pltpu.VMEM((STRIPE,), jnp.float32),
            pltpu.VMEM_SHARED((NPAD,), jnp.float32),
        ],
    )
    return f(scores, vv)


def _msg_body(msrc, mdst, emsg, uu, vv, ex, spart, opart,
              s_tab, t_tab, idxu, idxv, rows_u, rows_v, rows_e, exbuf,
              zbuf, acc, semu, semv, seme):
    cid = lax.axis_index("c")
    sid = lax.axis_index("s")
    wid = sid * NC + cid
    base = wid * EPW
    QS = STRIPE // 4

    pltpu.sync_copy(spart.at[0], s_tab)
    pltpu.sync_copy(spart.at[1], t_tab)

    def add_body(i, c):
        sl = pl.ds(i * 16, 16)
        s_tab[sl] = s_tab[sl] + t_tab[sl]
        return c

    lax.fori_loop(0, NPAD // 16, add_body, 0)

    def z_body(i, c):
        for d in range(D // 16):
            zbuf[i, pl.ds(d * 16, 16)] = jnp.zeros((16,), jnp.float32)
        return c

    lax.fori_loop(0, QS, z_body, 0)
    for k in range(4):
        pltpu.sync_copy(zbuf, acc.at[pl.ds(sid * STRIPE + k * QS, QS)])
    plsc.subcore_barrier()

    def blk_body(b, carry):
        off = base + b * BLK
        pltpu.sync_copy(uu.at[pl.ds(off, BLK)], idxu)
        pltpu.sync_copy(vv.at[pl.ds(off, BLK)], idxv)
        pltpu.sync_copy(ex.at[pl.ds(off, BLK)], exbuf)
        cu = pltpu.async_copy(msrc.at[idxu], rows_u, semu)
        cv = pltpu.async_copy(mdst.at[idxv], rows_v, semv)
        ce = pltpu.async_copy(emsg.at[pl.ds(off, BLK)], rows_e, seme)
        cu.wait()
        cv.wait()
        ce.wait()

        def a_body(g, c2):
            sl = pl.ds(g * 16, 16)
            dsum = plsc.load_gather(s_tab, [idxv[sl]])
            exbuf[sl] = exbuf[sl] / dsum
            return c2

        lax.fori_loop(0, BLK // 16, a_body, 0)

        def e_body(e, c2):
            a = exbuf[e]
            for d in range(D // 16):
                sl = pl.ds(d * 16, 16)
                x = rows_u[e, sl] + rows_v[e, sl] + rows_e[e, sl]
                rows_e[e, sl] = _lrelu(x) * a
            return c2

        lax.fori_loop(0, BLK, e_body, 0)
        pltpu.sync_copy(rows_e, acc.at[idxv], add=True)
        return carry

    lax.fori_loop(0, NBLK, blk_body, 0)
    plsc.subcore_barrier()
    for k in range(4):
        off = sid * STRIPE + k * QS
        pltpu.sync_copy(acc.at[pl.ds(off, QS)], zbuf)
        pltpu.sync_copy(zbuf, opart.at[cid, pl.ds(off, QS)])


def _messages(msrc, mdst, emsg, uu, vv, ex, spart):
    f = pl.kernel(
        _msg_body,
        out_type=jax.ShapeDtypeStruct((NC, NPAD, D), jnp.float32),
        mesh=_MESH,
        scratch_types=[
            pltpu.VMEM((NPAD,), jnp.float32),
            pltpu.VMEM((NPAD,), jnp.float32),
            pltpu.VMEM((BLK,), jnp.int32),
            pltpu.VMEM((BLK,), jnp.int32),
            pltpu.VMEM((BLK, D), jnp.float32),
            pltpu.VMEM((BLK, D), jnp.float32),
            pltpu.VMEM((BLK, D), jnp.float32),
            pltpu.VMEM((BLK,), jnp.float32),
            pltpu.VMEM((STRIPE // 4, D), jnp.float32),
            pltpu.VMEM_SHARED((NPAD, D), jnp.float32),
            pltpu.SemaphoreType.DMA,
            pltpu.SemaphoreType.DMA,
            pltpu.SemaphoreType.DMA,
        ],
    )
    return f(msrc, mdst, emsg, uu, vv, ex, spart)


@jax.jit
def kernel(node_feats, edge_feats, edge_index,
           W_attn_src, b_attn_src, W_attn_dst, b_attn_dst,
           W_attn_edg, b_attn_edg, W_attn_dot, b_attn_dot,
           W_msg_src, b_msg_src, W_msg_dst, b_msg_dst,
           W_msg_edg, b_msg_edg, W_wgt_n, b_wgt_n):
    nf2d = node_feats.reshape(N, D)
    uu = edge_index[0]
    vv = edge_index[1]

    wcat = jnp.concatenate(
        [W_attn_src, W_attn_dst, W_msg_src, W_msg_dst], axis=0).T
    wecat = jnp.concatenate([W_attn_edg, W_msg_edg], axis=0).T
    becat = jnp.concatenate(
        [b_attn_src + b_attn_dst + b_attn_edg,
         b_msg_src + b_msg_dst + b_msg_edg]).reshape(1, 2 * D)
    wdot = W_attn_dot[0]
    bn = b_wgt_n.reshape(1, D)

    asrc, adst, msrc, mdst = _node_transform(nf2d, wcat)
    eatt, emsg = _edge_transform(edge_feats, wecat, becat)

    scores = _scores(asrc, adst, eatt, uu, vv, wdot)
    ex, spart = _denoms(scores, vv)
    opart = _messages(msrc, mdst, emsg, uu, vv, ex, spart)

    out = _final_combine(opart[0, :N, :], opart[1, :N, :], nf2d,
                         W_wgt_n.T, bn)
    return out.reshape(N, 1, D)


# speedup vs baseline: 5.1126x; 5.1126x over previous
import functools

import jax
import jax.numpy as jnp
from jax import lax
from jax.experimental import pallas as pl
from jax.experimental.pallas import tpu as pltpu
from jax.experimental.pallas import tpu_sc as plsc

N = 10000
NPAD = 10240
E = 320000
D = 128
DE = 16
NC = 2
NS = 16
NW = NC * NS
EPW = E // NW
BLK = 80
NBLK = EPW // BLK
MBLK = 80
NMBLK = EPW // MBLK
CBLK = 2000
NCBLK = EPW // CBLK
STRIPE = NPAD // NS


def _lrelu(x):
    return jnp.where(x > 0, x, 0.2 * x)


def _node_mm_body(x_ref, w_ref, o0, o1, o2, o3):
    y = jnp.dot(x_ref[:], w_ref[:], preferred_element_type=jnp.float32)
    o0[:] = y[:, 0 * D:1 * D]
    o1[:] = y[:, 1 * D:2 * D]
    o2[:] = y[:, 2 * D:3 * D]
    o3[:] = y[:, 3 * D:4 * D]


def _node_transform(nf2d, wcat):
    blk = 2000
    sds = jax.ShapeDtypeStruct((N, D), jnp.float32)
    return pl.pallas_call(
        _node_mm_body,
        grid=(N // blk,),
        in_specs=[
            pl.BlockSpec((blk, D), lambda i: (i, 0)),
            pl.BlockSpec((D, 4 * D), lambda i: (0, 0)),
        ],
        out_specs=[pl.BlockSpec((blk, D), lambda i: (i, 0))] * 4,
        out_shape=[sds] * 4,
    )(nf2d, wcat)


def _edge_mm_body(x_ref, w_ref, b_ref, o0, o1):
    y = jnp.dot(x_ref[:], w_ref[:], preferred_element_type=jnp.float32)
    y = y + b_ref[:]
    o0[:] = y[:, 0 * D:1 * D]
    o1[:] = y[:, 1 * D:2 * D]


def _edge_transform(ef, wecat, becat):
    blk = 4000
    sds = jax.ShapeDtypeStruct((E, D), jnp.float32)
    return pl.pallas_call(
        _edge_mm_body,
        grid=(E // blk,),
        in_specs=[
            pl.BlockSpec((blk, DE), lambda i: (i, 0)),
            pl.BlockSpec((DE, 2 * D), lambda i: (0, 0)),
            pl.BlockSpec((1, 2 * D), lambda i: (0, 0)),
        ],
        out_specs=[pl.BlockSpec((blk, D), lambda i: (i, 0))] * 2,
        out_shape=[sds] * 2,
    )(ef, wecat, becat)


def _final_body(p0_ref, p1_ref, x_ref, w_ref, b_ref, o_ref):
    y = jnp.dot(x_ref[:], w_ref[:], preferred_element_type=jnp.float32)
    y = y + b_ref[:] + p0_ref[:] + p1_ref[:]
    o_ref[:] = _lrelu(y)


def _final_combine(p0, p1, nf2d, wnT, bn):
    blk = 2000
    return pl.pallas_call(
        _final_body,
        grid=(N // blk,),
        in_specs=[
            pl.BlockSpec((blk, D), lambda i: (i, 0)),
            pl.BlockSpec((blk, D), lambda i: (i, 0)),
            pl.BlockSpec((blk, D), lambda i: (i, 0)),
            pl.BlockSpec((D, D), lambda i: (0, 0)),
            pl.BlockSpec((1, D), lambda i: (0, 0)),
        ],
        out_specs=pl.BlockSpec((blk, D), lambda i: (i, 0)),
        out_shape=jax.ShapeDtypeStruct((N, D), jnp.float32),
    )(p0, p1, nf2d, wnT, bn)


_MESH = plsc.VectorSubcoreMesh(core_axis_name="c", subcore_axis_name="s")


def _worker_id():
    return lax.axis_index("s") * NC + lax.axis_index("c")


def _score_body(asrc, adst, eatt, uu, vv, wdot, scores,
                idxu, idxv, rows_u, rows_v, rows_e, wv, sbuf, abuf,
                semu, semv, seme):
    base = _worker_id() * EPW
    pltpu.sync_copy(wdot, wv)

    def blk_body(b, carry):
        off = base + b * BLK
        pltpu.sync_copy(uu.at[pl.ds(off, BLK)], idxu)
        pltpu.sync_copy(vv.at[pl.ds(off, BLK)], idxv)
        cu = pltpu.async_copy(asrc.at[idxu], rows_u, semu)
        cv = pltpu.async_copy(adst.at[idxv], rows_v, semv)
        ce = pltpu.async_copy(eatt.at[pl.ds(off, BLK)], rows_e, seme)
        cu.wait()
        cv.wait()
        ce.wait()

        def e_body(e, c2):
            acc = jnp.zeros((16,), jnp.float32)
            for d in range(D // 16):
                sl = pl.ds(d * 16, 16)
                x = rows_u[e, sl] + rows_v[e, sl] + rows_e[e, sl]
                acc = acc + wv[sl] * _lrelu(x)
            abuf[pl.ds(e * 16, 16)] = acc
            return c2

        lax.fori_loop(0, BLK, e_body, 0)

        def h_body(g, c2):
            rows16 = (lax.iota(jnp.int32, 16) + g * 16) * 16
            tot = jnp.zeros((16,), jnp.float32)
            for l in range(16):
                tot = tot + plsc.load_gather(abuf, [rows16 + l])
            sbuf[pl.ds(g * 16, 16)] = tot
            return c2

        lax.fori_loop(0, BLK // 16, h_body, 0)
        pltpu.sync_copy(sbuf, scores.at[pl.ds(off, BLK)])
        return carry

    lax.fori_loop(0, NBLK, blk_body, 0)


def _scores(asrc, adst, eatt, uu, vv, wdot):
    f = pl.kernel(
        _score_body,
        out_type=jax.ShapeDtypeStruct((E,), jnp.float32),
        mesh=_MESH,
        compiler_params=pltpu.CompilerParams(needs_layout_passes=False),
        scratch_types=[
            pltpu.VMEM((BLK,), jnp.int32),
            pltpu.VMEM((BLK,), jnp.int32),
            pltpu.VMEM((BLK, D), jnp.float32),
            pltpu.VMEM((BLK, D), jnp.float32),
            pltpu.VMEM((BLK, D), jnp.float32),
            pltpu.VMEM((D,), jnp.float32),
            pltpu.VMEM((BLK,), jnp.float32),
            pltpu.VMEM((BLK * 16,), jnp.float32),
            pltpu.SemaphoreType.DMA,
            pltpu.SemaphoreType.DMA,
            pltpu.SemaphoreType.DMA,
        ],
    )
    return f(asrc, adst, eatt, uu, vv, wdot)


def _denom_body(scores, vv, ex, spart,
                vbuf, sbuf, ebuf, zbuf, obuf, acc):
    cid = lax.axis_index("c")
    sid = lax.axis_index("s")
    wid = sid * NC + cid
    base = wid * EPW

    def z_body(i, c):
        zbuf[pl.ds(i * 16, 16)] = jnp.zeros((16,), jnp.float32)
        return c

    lax.fori_loop(0, STRIPE // 16, z_body, 0)
    pltpu.sync_copy(zbuf, acc.at[pl.ds(sid * STRIPE, STRIPE)])
    plsc.subcore_barrier()

    def blk_body(b, carry):
        off = base + b * CBLK
        pltpu.sync_copy(vv.at[pl.ds(off, CBLK)], vbuf)
        pltpu.sync_copy(scores.at[pl.ds(off, CBLK)], sbuf)

        def g_body(g, c2):
            sl = pl.ds(g * 16, 16)
            ebuf[sl] = jnp.exp(sbuf[sl])
            return c2

        lax.fori_loop(0, CBLK // 16, g_body, 0)
        pltpu.sync_copy(ebuf, ex.at[pl.ds(off, CBLK)])
        pltpu.sync_copy(ebuf, acc.at[vbuf], add=True)
        return carry

    lax.fori_loop(0, NCBLK, blk_body, 0)
    plsc.subcore_barrier()
    pltpu.sync_copy(acc.at[pl.ds(sid * STRIPE, STRIPE)], obuf)
    pltpu.sync_copy(obuf, spart.at[cid, pl.ds(sid * STRIPE, STRIPE)])


def _denoms(scores, vv):
    f = pl.kernel(
        _denom_body,
        out_type=(
            jax.ShapeDtypeStruct((E,), jnp.float32),
            jax.ShapeDtypeStruct((NC, NPAD), jnp.float32),
        ),
        mesh=_MESH,
        compiler_params=pltpu.CompilerParams(needs_layout_passes=False),
        scratch_types=[
            pltpu.VMEM((CBLK,), jnp.int32),
            pltpu.VMEM((CBLK,), jnp.float32),
            pltpu.VMEM((CBLK,), jnp.float32),
            pltpu.VMEM((STRIPE,), jnp.float32),
            pltpu.VMEM((STRIPE,), jnp.float32),
            pltpu.VMEM_SHARED((NPAD,), jnp.float32),
        ],
    )
    return f(scores, vv)


def _msg_body(msrc, mdst, emsg, uu, vv, ex, spart, opart,
              s_tab, stg, idxu, idxv, rows_u, rows_v, rows_e, exbuf,
              acc, semu, semv, seme):
    cid = lax.axis_index("c")
    sid = lax.axis_index("s")
    wid = sid * NC + cid
    base = wid * EPW

    pltpu.sync_copy(spart.at[0], s_tab)
    for k in range(NPAD // 1280):
        pltpu.sync_copy(spart.at[1, pl.ds(k * 1280, 1280)], stg)

        def add_body(i, c, k=k):
            s_tab[pl.ds(k * 1280 + i * 16, 16)] = (
                s_tab[pl.ds(k * 1280 + i * 16, 16)] + stg[pl.ds(i * 16, 16)])
            return c

        lax.fori_loop(0, 1280 // 16, add_body, 0)

    def z_body(i, c):
        for d in range(D // 16):
            rows_u[i, pl.ds(d * 16, 16)] = jnp.zeros((16,), jnp.float32)
        return c

    lax.fori_loop(0, MBLK, z_body, 0)
    for k in range(STRIPE // MBLK):
        pltpu.sync_copy(rows_u, acc.at[pl.ds(sid * STRIPE + k * MBLK, MBLK)])
    plsc.subcore_barrier()

    def blk_body(b, carry):
        off = base + b * MBLK
        pltpu.sync_copy(uu.at[pl.ds(off, MBLK)], idxu)
        pltpu.sync_copy(vv.at[pl.ds(off, MBLK)], idxv)
        pltpu.sync_copy(ex.at[pl.ds(off, MBLK)], exbuf)
        cu = pltpu.async_copy(msrc.at[idxu], rows_u, semu)
        cv = pltpu.async_copy(mdst.at[idxv], rows_v, semv)
        ce = pltpu.async_copy(emsg.at[pl.ds(off, MBLK)], rows_e, seme)
        cu.wait()
        cv.wait()
        ce.wait()

        def a_body(g, c2):
            sl = pl.ds(g * 16, 16)
            dsum = plsc.load_gather(s_tab, [idxv[sl]])
            exbuf[sl] = exbuf[sl] / dsum
            return c2

        lax.fori_loop(0, MBLK // 16, a_body, 0)

        def e_body(g, c2):
            al16 = exbuf[pl.ds(g * 16, 16)]
            for j in range(16):
                e = g * 16 + j
                a = al16[j]
                for d in range(D // 16):
                    sl = pl.ds(d * 16, 16)
                    x = rows_u[e, sl] + rows_v[e, sl] + rows_e[e, sl]
                    rows_e[e, sl] = _lrelu(x) * a
            return c2

        lax.fori_loop(0, MBLK // 16, e_body, 0)
        pltpu.sync_copy(rows_e, acc.at[idxv], add=True)
        return carry

    lax.fori_loop(0, NMBLK, blk_body, 0)
    plsc.subcore_barrier()
    for k in range(STRIPE // MBLK):
        off = sid * STRIPE + k * MBLK
        pltpu.sync_copy(acc.at[pl.ds(off, MBLK)], rows_u)
        pltpu.sync_copy(rows_u, opart.at[cid, pl.ds(off, MBLK)])


def _messages(msrc, mdst, emsg, uu, vv, ex, spart):
    f = pl.kernel(
        _msg_body,
        out_type=jax.ShapeDtypeStruct((NC, NPAD, D), jnp.float32),
        mesh=_MESH,
        compiler_params=pltpu.CompilerParams(needs_layout_passes=False),
        scratch_types=[
            pltpu.VMEM((NPAD,), jnp.float32),
            pltpu.VMEM((1280,), jnp.float32),
            pltpu.VMEM((MBLK,), jnp.int32),
            pltpu.VMEM((MBLK,), jnp.int32),
            pltpu.VMEM((MBLK, D), jnp.float32),
            pltpu.VMEM((MBLK, D), jnp.float32),
            pltpu.VMEM((MBLK, D), jnp.float32),
            pltpu.VMEM((MBLK,), jnp.float32),
            pltpu.VMEM_SHARED((NPAD, D), jnp.float32),
            pltpu.SemaphoreType.DMA,
            pltpu.SemaphoreType.DMA,
            pltpu.SemaphoreType.DMA,
        ],
    )
    return f(msrc, mdst, emsg, uu, vv, ex, spart)


@jax.jit
def kernel(node_feats, edge_feats, edge_index,
           W_attn_src, b_attn_src, W_attn_dst, b_attn_dst,
           W_attn_edg, b_attn_edg, W_attn_dot, b_attn_dot,
           W_msg_src, b_msg_src, W_msg_dst, b_msg_dst,
           W_msg_edg, b_msg_edg, W_wgt_n, b_wgt_n):
    nf2d = node_feats.reshape(N, D)
    uu = edge_index[0]
    vv = edge_index[1]

    wcat = jnp.concatenate(
        [W_attn_src, W_attn_dst, W_msg_src, W_msg_dst], axis=0).T
    wecat = jnp.concatenate([W_attn_edg, W_msg_edg], axis=0).T
    becat = jnp.concatenate(
        [b_attn_src + b_attn_dst + b_attn_edg,
         b_msg_src + b_msg_dst + b_msg_edg]).reshape(1, 2 * D)
    wdot = W_attn_dot[0]
    bn = b_wgt_n.reshape(1, D)

    asrc, adst, msrc, mdst = _node_transform(nf2d, wcat)
    eatt, emsg = _edge_transform(edge_feats, wecat, becat)

    scores = _scores(asrc, adst, eatt, uu, vv, wdot)
    ex, spart = _denoms(scores, vv)
    opart = _messages(msrc, mdst, emsg, uu, vv, ex, spart)

    out = _final_combine(opart[0, :N, :], opart[1, :N, :], nf2d,
                         W_wgt_n.T, bn)
    return out.reshape(N, 1, D)


# speedup vs baseline: 5.8848x; 1.1510x over previous
import jax
import jax.numpy as jnp
from jax import lax
from jax.experimental import pallas as pl
from jax.experimental.pallas import tpu as pltpu
from jax.experimental.pallas import tpu_sc as plsc

N = 10000
NPAD = 10240
ACCN = 10112
E = 320000
D = 128
DE = 16
NC = 2
NS = 16
NW = NC * NS
EPW = E // NW
ABLK = 64
ANFULL = 156
ANPAIR = ANFULL // 2
MBLK = 48
MNFULL = 208
MNPAIR = MNFULL // 2
TAIL = 16
ATOFF = ANFULL * ABLK
MTOFF = MNFULL * MBLK
CBLK = 2000
DSTRIPE = NPAD // NS
ASTRIPE = ACCN // NS


def _lrelu(x):
    return jnp.where(x > 0, x, 0.2 * x)


def _mm2_body(x_ref, w_ref, o0, o1):
    y = jnp.dot(x_ref[:], w_ref[:], preferred_element_type=jnp.float32)
    o0[:] = y[:, 0 * D:1 * D]
    o1[:] = y[:, 1 * D:2 * D]


def _node_transform2(nf2d, wcat):
    blk = 2000
    sds = jax.ShapeDtypeStruct((N, D), jnp.float32)
    return pl.pallas_call(
        _mm2_body,
        grid=(N // blk,),
        in_specs=[
            pl.BlockSpec((blk, D), lambda i: (i, 0)),
            pl.BlockSpec((D, 2 * D), lambda i: (0, 0)),
        ],
        out_specs=[pl.BlockSpec((blk, D), lambda i: (i, 0))] * 2,
        out_shape=[sds] * 2,
    )(nf2d, wcat)


def _emm_body(x_ref, w_ref, b_ref, o_ref):
    y = jnp.dot(x_ref[:], w_ref[:], preferred_element_type=jnp.float32)
    o_ref[:] = y + b_ref[:]


def _edge_transform1(ef, w, b):
    blk = 4000
    return pl.pallas_call(
        _emm_body,
        grid=(E // blk,),
        in_specs=[
            pl.BlockSpec((blk, DE), lambda i: (i, 0)),
            pl.BlockSpec((DE, D), lambda i: (0, 0)),
            pl.BlockSpec((1, D), lambda i: (0, 0)),
        ],
        out_specs=pl.BlockSpec((blk, D), lambda i: (i, 0)),
        out_shape=jax.ShapeDtypeStruct((E, D), jnp.float32),
    )(ef, w, b)


def _final_body(p0_ref, p1_ref, x_ref, w_ref, b_ref, o_ref):
    y = jnp.dot(x_ref[:], w_ref[:], preferred_element_type=jnp.float32)
    y = y + b_ref[:] + p0_ref[:] + p1_ref[:]
    o_ref[:] = _lrelu(y)


def _final_combine(p0, p1, nf2d, wnT, bn):
    blk = 2000
    return pl.pallas_call(
        _final_body,
        grid=(N // blk,),
        in_specs=[
            pl.BlockSpec((blk, D), lambda i: (i, 0)),
            pl.BlockSpec((blk, D), lambda i: (i, 0)),
            pl.BlockSpec((blk, D), lambda i: (i, 0)),
            pl.BlockSpec((D, D), lambda i: (0, 0)),
            pl.BlockSpec((1, D), lambda i: (0, 0)),
        ],
        out_specs=pl.BlockSpec((blk, D), lambda i: (i, 0)),
        out_shape=jax.ShapeDtypeStruct((N, D), jnp.float32),
    )(p0, p1, nf2d, wnT, bn)


_MESH = plsc.VectorSubcoreMesh(core_axis_name="c", subcore_axis_name="s")
_SC_PARAMS = pltpu.CompilerParams(needs_layout_passes=False)


def _score_body(asrc, adst, eatt, uu, vv, wdot, ex, spart,
                wv, idxu0, idxv0, idxu1, idxv1, idxut, idxvt,
                ru0, rv0, re0, ru1, rv1, re1, sbuf, abuf, zbuf, dacc,
                su0, sv0, se0, su1, sv1, se1):
    cid = lax.axis_index("c")
    sid = lax.axis_index("s")
    base = (sid * NC + cid) * EPW
    pltpu.sync_copy(wdot, wv)

    def z_body(i, c):
        zbuf[pl.ds(i * 16, 16)] = jnp.zeros((16,), jnp.float32)
        return c

    lax.fori_loop(0, DSTRIPE // 16, z_body, 0)
    pltpu.sync_copy(zbuf, dacc.at[pl.ds(sid * DSTRIPE, DSTRIPE)])
    plsc.subcore_barrier()

    def issue(off, idxu, idxv, ru, rv, re, su, sv, se):
        pltpu.sync_copy(uu.at[pl.ds(off, ABLK)], idxu)
        pltpu.sync_copy(vv.at[pl.ds(off, ABLK)], idxv)
        pltpu.async_copy(asrc.at[idxu], ru, su)
        pltpu.async_copy(adst.at[idxv], rv, sv)
        pltpu.async_copy(eatt.at[pl.ds(off, ABLK)], re, se)

    def wait(off, idxu, idxv, ru, rv, re, su, sv, se):
        pltpu.make_async_copy(asrc.at[idxu], ru, su).wait()
        pltpu.make_async_copy(adst.at[idxv], rv, sv).wait()
        pltpu.make_async_copy(eatt.at[pl.ds(off, ABLK)], re, se).wait()

    def compute(off, idxv, ru, rv, re, nb):
        def e_body(e, c2):
            acc = jnp.zeros((16,), jnp.float32)
            for d in range(D // 16):
                sl = pl.ds(d * 16, 16)
                x = ru[e, sl] + rv[e, sl] + re[e, sl]
                acc = acc + wv[sl] * _lrelu(x)
            abuf[pl.ds(e * 16, 16)] = acc
            return c2

        lax.fori_loop(0, nb, e_body, 0)

        def h_body(g, c2):
            rows16 = (lax.iota(jnp.int32, 16) + g * 16) * 16
            tot = jnp.zeros((16,), jnp.float32)
            for l in range(16):
                tot = tot + plsc.load_gather(abuf, [rows16 + l])
            sbuf[pl.ds(g * 16, 16)] = jnp.exp(tot)
            return c2

        lax.fori_loop(0, nb // 16, h_body, 0)
        if nb == ABLK:
            pltpu.sync_copy(sbuf, ex.at[pl.ds(off, ABLK)])
            pltpu.sync_copy(sbuf, dacc.at[idxv], add=True)
        else:
            pltpu.sync_copy(sbuf.at[pl.ds(0, TAIL)], ex.at[pl.ds(off, TAIL)])
            pltpu.sync_copy(sbuf.at[pl.ds(0, TAIL)], dacc.at[idxv], add=True)

    issue(base, idxu0, idxv0, ru0, rv0, re0, su0, sv0, se0)

    def pair(i, c):
        offa = base + (2 * i) * ABLK
        offb = offa + ABLK
        issue(offb, idxu1, idxv1, ru1, rv1, re1, su1, sv1, se1)
        wait(offa, idxu0, idxv0, ru0, rv0, re0, su0, sv0, se0)
        compute(offa, idxv0, ru0, rv0, re0, ABLK)
        offc = offb + ABLK

        @pl.when(2 * i + 2 < ANFULL)
        def _():
            issue(offc, idxu0, idxv0, ru0, rv0, re0, su0, sv0, se0)

        wait(offb, idxu1, idxv1, ru1, rv1, re1, su1, sv1, se1)
        compute(offb, idxv1, ru1, rv1, re1, ABLK)
        return c

    lax.fori_loop(0, ANPAIR, pair, 0)

    offt = base + ATOFF
    pltpu.sync_copy(uu.at[pl.ds(offt, TAIL)], idxut)
    pltpu.sync_copy(vv.at[pl.ds(offt, TAIL)], idxvt)
    cu = pltpu.async_copy(asrc.at[idxut], ru0.at[pl.ds(0, TAIL)], su0)
    cv = pltpu.async_copy(adst.at[idxvt], rv0.at[pl.ds(0, TAIL)], sv0)
    ce = pltpu.async_copy(eatt.at[pl.ds(offt, TAIL)], re0.at[pl.ds(0, TAIL)],
                          se0)
    cu.wait()
    cv.wait()
    ce.wait()
    compute(offt, idxvt, ru0, rv0, re0, TAIL)

    plsc.subcore_barrier()
    pltpu.sync_copy(dacc.at[pl.ds(sid * DSTRIPE, DSTRIPE)], zbuf)
    pltpu.sync_copy(zbuf, spart.at[cid, pl.ds(sid * DSTRIPE, DSTRIPE)])


def _scores(asrc, adst, eatt, uu, vv, wdot):
    f = pl.kernel(
        _score_body,
        out_type=(jax.ShapeDtypeStruct((E,), jnp.float32),
                  jax.ShapeDtypeStruct((NC, NPAD), jnp.float32)),
        mesh=_MESH,
        compiler_params=_SC_PARAMS,
        scratch_types=[
            pltpu.VMEM((D,), jnp.float32),
            pltpu.VMEM((ABLK,), jnp.int32),
            pltpu.VMEM((ABLK,), jnp.int32),
            pltpu.VMEM((ABLK,), jnp.int32),
            pltpu.VMEM((ABLK,), jnp.int32),
            pltpu.VMEM((TAIL,), jnp.int32),
            pltpu.VMEM((TAIL,), jnp.int32),
            pltpu.VMEM((ABLK, D), jnp.float32),
            pltpu.VMEM((ABLK, D), jnp.float32),
            pltpu.VMEM((ABLK, D), jnp.float32),
            pltpu.VMEM((ABLK, D), jnp.float32),
            pltpu.VMEM((ABLK, D), jnp.float32),
            pltpu.VMEM((ABLK, D), jnp.float32),
            pltpu.VMEM((ABLK,), jnp.float32),
            pltpu.VMEM((ABLK * 16,), jnp.float32),
            pltpu.VMEM((DSTRIPE,), jnp.float32),
            pltpu.VMEM_SHARED((NPAD,), jnp.float32),
            pltpu.SemaphoreType.DMA,
            pltpu.SemaphoreType.DMA,
            pltpu.SemaphoreType.DMA,
            pltpu.SemaphoreType.DMA,
            pltpu.SemaphoreType.DMA,
            pltpu.SemaphoreType.DMA,
        ],
    )
    return f(asrc, adst, eatt, uu, vv, wdot)


def _alpha_body(ex, vv, spart, alpha, t0, t1, vb, eb):
    cid = lax.axis_index("c")
    sid = lax.axis_index("s")
    base = (sid * NC + cid) * EPW
    pltpu.sync_copy(spart.at[0], t0)
    pltpu.sync_copy(spart.at[1], t1)

    def blk_body(b, c):
        off = base + b * CBLK
        pltpu.sync_copy(vv.at[pl.ds(off, CBLK)], vb)
        pltpu.sync_copy(ex.at[pl.ds(off, CBLK)], eb)

        def g_body(g, c2):
            sl = pl.ds(g * 16, 16)
            v16 = vb[sl]
            dsum = plsc.load_gather(t0, [v16]) + plsc.load_gather(t1, [v16])
            eb[sl] = eb[sl] / dsum
            return c2

        lax.fori_loop(0, CBLK // 16, g_body, 0)
        pltpu.sync_copy(eb, alpha.at[pl.ds(off, CBLK)])
        return c

    lax.fori_loop(0, EPW // CBLK, blk_body, 0)


def _alphas(ex, vv, spart):
    f = pl.kernel(
        _alpha_body,
        out_type=jax.ShapeDtypeStruct((E,), jnp.float32),
        mesh=_MESH,
        compiler_params=_SC_PARAMS,
        scratch_types=[
            pltpu.VMEM((NPAD,), jnp.float32),
            pltpu.VMEM((NPAD,), jnp.float32),
            pltpu.VMEM((CBLK,), jnp.int32),
            pltpu.VMEM((CBLK,), jnp.float32),
        ],
    )
    return f(ex, vv, spart)


def _msg_body(msrc, mdst, emsg, uu, vv, alpha, opart,
              idxu0, idxv0, idxu1, idxv1, idxut, idxvt,
              ru0, rv0, re0, ru1, rv1, re1, alb0, alb1, acc,
              su0, sv0, se0, su1, sv1, se1):
    cid = lax.axis_index("c")
    sid = lax.axis_index("s")
    base = (sid * NC + cid) * EPW

    def z_body(i, c):
        for d in range(D // 16):
            ru0[i, pl.ds(d * 16, 16)] = jnp.zeros((16,), jnp.float32)
        return c

    lax.fori_loop(0, MBLK, z_body, 0)
    for k in range(ASTRIPE // MBLK):
        pltpu.sync_copy(ru0, acc.at[pl.ds(sid * ASTRIPE + k * MBLK, MBLK)])
    rem = ASTRIPE % MBLK
    pltpu.sync_copy(ru0.at[pl.ds(0, rem)],
                    acc.at[pl.ds(sid * ASTRIPE + ASTRIPE - rem, rem)])
    plsc.subcore_barrier()

    def issue(off, idxu, idxv, ru, rv, re, alb, su, sv, se):
        pltpu.sync_copy(uu.at[pl.ds(off, MBLK)], idxu)
        pltpu.sync_copy(vv.at[pl.ds(off, MBLK)], idxv)
        pltpu.sync_copy(alpha.at[pl.ds(off, MBLK)], alb)
        pltpu.async_copy(msrc.at[idxu], ru, su)
        pltpu.async_copy(mdst.at[idxv], rv, sv)
        pltpu.async_copy(emsg.at[pl.ds(off, MBLK)], re, se)

    def wait(off, idxu, idxv, ru, rv, re, su, sv, se):
        pltpu.make_async_copy(msrc.at[idxu], ru, su).wait()
        pltpu.make_async_copy(mdst.at[idxv], rv, sv).wait()
        pltpu.make_async_copy(emsg.at[pl.ds(off, MBLK)], re, se).wait()

    def compute(idxv, ru, rv, re, alb, nb):
        def e_body(g, c2):
            al16 = alb[pl.ds(g * 16, 16)]
            for j in range(16):
                e = g * 16 + j
                a = al16[j]
                for d in range(D // 16):
                    sl = pl.ds(d * 16, 16)
                    x = ru[e, sl] + rv[e, sl] + re[e, sl]
                    re[e, sl] = _lrelu(x) * a
            return c2

        lax.fori_loop(0, nb // 16, e_body, 0)
        if nb == MBLK:
            pltpu.sync_copy(re, acc.at[idxv], add=True)
        else:
            pltpu.sync_copy(re.at[pl.ds(0, TAIL)], acc.at[idxv], add=True)

    issue(base, idxu0, idxv0, ru0, rv0, re0, alb0, su0, sv0, se0)

    def pair(i, c):
        offa = base + (2 * i) * MBLK
        offb = offa + MBLK
        issue(offb, idxu1, idxv1, ru1, rv1, re1, alb1, su1, sv1, se1)
        wait(offa, idxu0, idxv0, ru0, rv0, re0, su0, sv0, se0)
        compute(idxv0, ru0, rv0, re0, alb0, MBLK)
        offc = offb + MBLK

        @pl.when(2 * i + 2 < MNFULL)
        def _():
            issue(offc, idxu0, idxv0, ru0, rv0, re0, alb0, su0, sv0, se0)

        wait(offb, idxu1, idxv1, ru1, rv1, re1, su1, sv1, se1)
        compute(idxv1, ru1, rv1, re1, alb1, MBLK)
        return c

    lax.fori_loop(0, MNPAIR, pair, 0)

    offt = base + MTOFF
    pltpu.sync_copy(uu.at[pl.ds(offt, TAIL)], idxut)
    pltpu.sync_copy(vv.at[pl.ds(offt, TAIL)], idxvt)
    pltpu.sync_copy(alpha.at[pl.ds(offt, TAIL)], alb0.at[pl.ds(0, TAIL)])
    cu = pltpu.async_copy(msrc.at[idxut], ru0.at[pl.ds(0, TAIL)], su0)
    cv = pltpu.async_copy(mdst.at[idxvt], rv0.at[pl.ds(0, TAIL)], sv0)
    ce = pltpu.async_copy(emsg.at[pl.ds(offt, TAIL)], re0.at[pl.ds(0, TAIL)],
                          se0)
    cu.wait()
    cv.wait()
    ce.wait()
    compute(idxvt, ru0, rv0, re0, alb0, TAIL)

    plsc.subcore_barrier()
    for k in range(ASTRIPE // MBLK):
        off = sid * ASTRIPE + k * MBLK
        pltpu.sync_copy(acc.at[pl.ds(off, MBLK)], ru0)
        pltpu.sync_copy(ru0, opart.at[cid, pl.ds(off, MBLK)])
    offr = sid * ASTRIPE + ASTRIPE - rem
    pltpu.sync_copy(acc.at[pl.ds(offr, rem)], ru0.at[pl.ds(0, rem)])
    pltpu.sync_copy(ru0.at[pl.ds(0, rem)], opart.at[cid, pl.ds(offr, rem)])


def _messages(msrc, mdst, emsg, uu, vv, alpha):
    f = pl.kernel(
        _msg_body,
        out_type=jax.ShapeDtypeStruct((NC, ACCN, D), jnp.float32),
        mesh=_MESH,
        compiler_params=_SC_PARAMS,
        scratch_types=[
            pltpu.VMEM((MBLK,), jnp.int32),
            pltpu.VMEM((MBLK,), jnp.int32),
            pltpu.VMEM((MBLK,), jnp.int32),
            pltpu.VMEM((MBLK,), jnp.int32),
            pltpu.VMEM((TAIL,), jnp.int32),
            pltpu.VMEM((TAIL,), jnp.int32),
            pltpu.VMEM((MBLK, D), jnp.float32),
            pltpu.VMEM((MBLK, D), jnp.float32),
            pltpu.VMEM((MBLK, D), jnp.float32),
            pltpu.VMEM((MBLK, D), jnp.float32),
            pltpu.VMEM((MBLK, D), jnp.float32),
            pltpu.VMEM((MBLK, D), jnp.float32),
            pltpu.VMEM((MBLK,), jnp.float32),
            pltpu.VMEM((MBLK,), jnp.float32),
            pltpu.VMEM_SHARED((ACCN, D), jnp.float32),
            pltpu.SemaphoreType.DMA,
            pltpu.SemaphoreType.DMA,
            pltpu.SemaphoreType.DMA,
            pltpu.SemaphoreType.DMA,
            pltpu.SemaphoreType.DMA,
            pltpu.SemaphoreType.DMA,
        ],
    )
    return f(msrc, mdst, emsg, uu, vv, alpha)


@jax.jit
def kernel(node_feats, edge_feats, edge_index,
           W_attn_src, b_attn_src, W_attn_dst, b_attn_dst,
           W_attn_edg, b_attn_edg, W_attn_dot, b_attn_dot,
           W_msg_src, b_msg_src, W_msg_dst, b_msg_dst,
           W_msg_edg, b_msg_edg, W_wgt_n, b_wgt_n):
    nf2d = node_feats.reshape(N, D)
    uu = edge_index[0]
    vv = edge_index[1]

    wcat_a = jnp.concatenate([W_attn_src, W_attn_dst], axis=0).T
    wcat_m = jnp.concatenate([W_msg_src, W_msg_dst], axis=0).T
    batt = (b_attn_src + b_attn_dst + b_attn_edg).reshape(1, D)
    bmsg = (b_msg_src + b_msg_dst + b_msg_edg).reshape(1, D)
    wdot = W_attn_dot[0]
    bn = b_wgt_n.reshape(1, D)

    asrc, adst = _node_transform2(nf2d, wcat_a)
    eatt = _edge_transform1(edge_feats, W_attn_edg.T, batt)
    msrc, mdst = _node_transform2(nf2d, wcat_m)
    emsg = _edge_transform1(edge_feats, W_msg_edg.T, bmsg)

    ex, spart = _scores(asrc, adst, eatt, uu, vv, wdot)
    alpha = _alphas(ex, vv, spart)
    opart = _messages(msrc, mdst, emsg, uu, vv, alpha)

    out = _final_combine(opart[0, :N, :], opart[1, :N, :], nf2d,
                         W_wgt_n.T, bn)
    return out.reshape(N, 1, D)
